# Initial kernel scaffold; baseline (speedup 1.0000x reference)
import jax
import jax.numpy as jnp
from jax.experimental import pallas as pl


def kernel(x, table):
    raise NotImplementedError("write your pallas kernel here")


# speedup vs baseline: 1.8580x; 1.8580x over previous
import functools
import jax
import jax.numpy as jnp
from jax import lax
from jax.experimental import pallas as pl
from jax.experimental.pallas import tpu as pltpu
from jax.experimental.pallas import tpu_sc as plsc

B = 4
L = 4096
D = 1024
NC = 2
NS = 16
LANES = 16
NW = NC * NS
NTOK = B * L
TOK_PER_W = NTOK // NW
CHUNK = 32
NCHUNK = TOK_PER_W // CHUNK


def _positional_encoding(seq_len, d_model):
    pos = jnp.arange(seq_len, dtype=jnp.float32)[:, None]
    _2i = jnp.arange(0, d_model, 2, dtype=jnp.float32)
    ang = pos / jnp.power(10000.0, _2i / d_model)
    pe = jnp.zeros((seq_len, d_model), dtype=jnp.float32)
    pe = pe.at[:, 0::2].set(jnp.sin(ang))
    pe = pe.at[:, 1::2].set(jnp.cos(ang))
    return pe


def _embed_body(x_hbm, table_hbm, pe_hbm, out_hbm, idx_v, rows_v, pe_v, sem):
    c = lax.axis_index("c")
    s = lax.axis_index("s")
    wid = s * NC + c
    base = wid * TOK_PER_W
    pos0 = lax.rem(base, L)
    pltpu.sync_copy(x_hbm.at[pl.ds(base, TOK_PER_W)], idx_v)

    def chunk_body(i, _):
        rbase = i * CHUNK
        gather = pltpu.async_copy(
            table_hbm.at[idx_v.at[pl.ds(rbase, CHUNK)]], rows_v, sem)
        pltpu.sync_copy(pe_hbm.at[pl.ds(pos0 + rbase, CHUNK)], pe_v)
        gather.wait()

        def row_body(j, _):
            for k in range(D // LANES):
                sl = pl.ds(k * LANES, LANES)
                rows_v[j, sl] = rows_v[j, sl] + pe_v[j, sl]
            return 0

        lax.fori_loop(0, CHUNK, row_body, 0, unroll=False)
        pltpu.sync_copy(rows_v, out_hbm.at[pl.ds(base + rbase, CHUNK)])
        return 0

    lax.fori_loop(0, NCHUNK, chunk_body, 0, unroll=False)


@functools.partial(
    pl.kernel,
    out_type=jax.ShapeDtypeStruct((NTOK, D), jnp.float32),
    mesh=plsc.VectorSubcoreMesh(core_axis_name="c", subcore_axis_name="s",
                                num_cores=NC, num_subcores=NS),
    scratch_types=[
        pltpu.VMEM((TOK_PER_W,), jnp.int32),
        pltpu.VMEM((CHUNK, D), jnp.float32),
        pltpu.VMEM((CHUNK, D), jnp.float32),
        pltpu.SemaphoreType.DMA,
    ],
)
def _sc_embed(x_hbm, table_hbm, pe_hbm, out_hbm, idx_v, rows_v, pe_v, sem):
    _embed_body(x_hbm, table_hbm, pe_hbm, out_hbm, idx_v, rows_v, pe_v, sem)


@jax.jit
def kernel(x, table):
    pe = _positional_encoding(L, D)
    xf = x.reshape(-1).astype(jnp.int32)
    out = _sc_embed(xf, table, pe)
    return out.reshape(B, L, D)


# speedup vs baseline: 2.3191x; 1.2482x over previous
import functools
import jax
import jax.numpy as jnp
from jax import lax
from jax.experimental import pallas as pl
from jax.experimental.pallas import tpu as pltpu
from jax.experimental.pallas import tpu_sc as plsc

B = 4
L = 4096
D = 1024
NC = 2
NS = 16
LANES = 16
NW = NC * NS
NTOK = B * L
TOK_PER_W = NTOK // NW
CHUNK = 16
NCHUNK = TOK_PER_W // CHUNK


def _positional_encoding(seq_len, d_model):
    pos = jnp.arange(seq_len, dtype=jnp.float32)[:, None]
    _2i = jnp.arange(0, d_model, 2, dtype=jnp.float32)
    ang = pos / jnp.power(10000.0, _2i / d_model)
    pe = jnp.zeros((seq_len, d_model), dtype=jnp.float32)
    pe = pe.at[:, 0::2].set(jnp.sin(ang))
    pe = pe.at[:, 1::2].set(jnp.cos(ang))
    return pe


def _embed_body(x_hbm, table_hbm, pe_hbm, out_hbm,
                idx_v, rows0, pe0, rows1, pe1, sg0, sg1, sw0, sw1):
    c = lax.axis_index("c")
    s = lax.axis_index("s")
    wid = s * NC + c
    base = wid * TOK_PER_W
    pos0 = lax.rem(base, L)
    pltpu.sync_copy(x_hbm.at[pl.ds(base, TOK_PER_W)], idx_v)

    bufs = ((rows0, pe0, sg0, sw0), (rows1, pe1, sg1, sw1))

    def in_copies(ci, rv, pv, sg):
        rbase = ci * CHUNK
        return (
            pltpu.make_async_copy(
                table_hbm.at[idx_v.at[pl.ds(rbase, CHUNK)]], rv, sg),
            pltpu.make_async_copy(
                pe_hbm.at[pl.ds(pos0 + rbase, CHUNK)], pv, sg),
        )

    def wb_copy(ci, rv, sw):
        return pltpu.make_async_copy(
            rv, out_hbm.at[pl.ds(base + ci * CHUNK, CHUNK)], sw)

    def add_chunk(rv, pv):
        def row_body(j, _):
            for k in range(D // LANES):
                sl = pl.ds(k * LANES, LANES)
                rv[j, sl] = rv[j, sl] + pv[j, sl]
            return 0
        lax.fori_loop(0, CHUNK, row_body, 0, unroll=False)

    for d in in_copies(0, rows0, pe0, sg0):
        d.start()

    def body(i, _):
        for p in range(2):
            rv, pv, sg, sw = bufs[p]
            nrv, npv, nsg, nsw = bufs[1 - p]
            ci = 2 * i + p
            @pl.when(ci >= 1)
            def _():
                wb_copy(ci - 1, nrv, nsw).wait()

            @pl.when(ci + 1 < NCHUNK)
            def _():
                for d in in_copies(ci + 1, nrv, npv, nsg):
                    d.start()
            for d in in_copies(ci, rv, pv, sg):
                d.wait()
            add_chunk(rv, pv)
            wb_copy(ci, rv, sw).start()
        return 0

    lax.fori_loop(0, NCHUNK // 2, body, 0, unroll=False)
    wb_copy(NCHUNK - 1, rows1, sw1).wait()


@functools.partial(
    pl.kernel,
    out_type=jax.ShapeDtypeStruct((NTOK, D), jnp.float32),
    mesh=plsc.VectorSubcoreMesh(core_axis_name="c", subcore_axis_name="s",
                                num_cores=NC, num_subcores=NS),
    scratch_types=[
        pltpu.VMEM((TOK_PER_W,), jnp.int32),
        pltpu.VMEM((CHUNK, D), jnp.float32),
        pltpu.VMEM((CHUNK, D), jnp.float32),
        pltpu.VMEM((CHUNK, D), jnp.float32),
        pltpu.VMEM((CHUNK, D), jnp.float32),
        pltpu.SemaphoreType.DMA,
        pltpu.SemaphoreType.DMA,
        pltpu.SemaphoreType.DMA,
        pltpu.SemaphoreType.DMA,
    ],
)
def _sc_embed(x_hbm, table_hbm, pe_hbm, out_hbm,
              idx_v, rows0, pe0, rows1, pe1, sg0, sg1, sw0, sw1):
    _embed_body(x_hbm, table_hbm, pe_hbm, out_hbm,
                idx_v, rows0, pe0, rows1, pe1, sg0, sg1, sw0, sw1)


@jax.jit
def kernel(x, table):
    pe = _positional_encoding(L, D)
    xf = x.reshape(-1).astype(jnp.int32)
    out = _sc_embed(xf, table, pe)
    return out.reshape(B, L, D)
